# Initial kernel scaffold; baseline (speedup 1.0000x reference)
import functools

import jax
import jax.numpy as jnp
from jax import lax
from jax.experimental import pallas as pl
from jax.experimental.pallas import tpu as pltpu
from jax.experimental.pallas import tpu_sc as plsc

N = 10000
F = 256
HALF = 128
E = 160000
NC = 2
NS = 16
DEG_W = 40
DEG_EPT = E // (NC * NS)
SC_W = 80
SC_EPT = E // NS
ROWS_PT = N // NS

_MESH = plsc.VectorSubcoreMesh(core_axis_name="c", subcore_axis_name="s")


@functools.partial(
    pl.kernel,
    out_type=jax.ShapeDtypeStruct((NC, N, 16), jnp.float32),
    mesh=_MESH,
    scratch_types=[
        pltpu.VMEM((DEG_W,), jnp.int32),
        pltpu.VMEM((DEG_W, 16), jnp.float32),
        pltpu.VMEM_SHARED((N, 16), jnp.float32),
    ],
)
def _sc_degree(dst_hbm, zeros_hbm, ones_hbm, out_hbm, dst_v, ones_v, deg_sh):
    cid = lax.axis_index("c")
    sid = lax.axis_index("s")
    wid = cid * NS + sid

    pltpu.sync_copy(ones_hbm, ones_v)

    @pl.when(sid == 0)
    def _zero():
        pltpu.sync_copy(zeros_hbm, deg_sh)

    plsc.subcore_barrier()

    def win(w, carry):
        base = wid * DEG_EPT + w * DEG_W
        pltpu.sync_copy(dst_hbm.at[pl.ds(base, DEG_W)], dst_v)
        pltpu.sync_copy(ones_v, deg_sh.at[dst_v], add=True)
        return carry

    lax.fori_loop(0, DEG_EPT // DEG_W, win, 0)
    plsc.subcore_barrier()

    @pl.when(sid == 0)
    def _out():
        pltpu.sync_copy(deg_sh, out_hbm.at[cid])


@functools.partial(
    pl.kernel,
    out_type=[
        jax.ShapeDtypeStruct((N, HALF), jnp.float32),
        jax.ShapeDtypeStruct((N, HALF), jnp.float32),
    ],
    mesh=_MESH,
    scratch_types=[
        pltpu.VMEM((SC_W,), jnp.int32),
        pltpu.VMEM((SC_W,), jnp.int32),
        pltpu.VMEM((SC_W, HALF), jnp.float32),
        pltpu.VMEM_SHARED((N, HALF), jnp.float32),
        pltpu.SemaphoreType.DMA,
    ],
)
def _sc_scatter(h0_hbm, h1_hbm, src_hbm, dst_hbm, zeros_hbm, out0_hbm,
                out1_hbm, src_v, dst_v, gath_v, acc_sh, sem):
    cid = lax.axis_index("c")
    sid = lax.axis_index("s")

    def run(h_hbm, out_hbm):
        pltpu.sync_copy(zeros_hbm.at[pl.ds(sid * ROWS_PT, ROWS_PT)],
                        acc_sh.at[pl.ds(sid * ROWS_PT, ROWS_PT)])
        plsc.subcore_barrier()

        def win(w, carry):
            base = sid * SC_EPT + w * SC_W
            pltpu.sync_copy(src_hbm.at[pl.ds(base, SC_W)], src_v)
            pltpu.sync_copy(dst_hbm.at[pl.ds(base, SC_W)], dst_v)
            pltpu.async_copy(h_hbm.at[src_v], gath_v, sem).wait()
            pltpu.sync_copy(gath_v, acc_sh.at[dst_v], add=True)
            return carry

        lax.fori_loop(0, SC_EPT // SC_W, win, 0)
        plsc.subcore_barrier()
        pltpu.sync_copy(acc_sh.at[pl.ds(sid * ROWS_PT, ROWS_PT)],
                        out_hbm.at[pl.ds(sid * ROWS_PT, ROWS_PT)])

    @pl.when(cid == 0)
    def _c0():
        run(h0_hbm, out0_hbm)

    @pl.when(cid == 1)
    def _c1():
        run(h1_hbm, out1_hbm)


def _tc1_body(degp_ref, z_ref, w1_ref, dinv_ref, h0_ref, h1_ref):
    deg = jnp.sum(degp_ref[...], axis=(0, 2)) + 1.0
    dinv = lax.rsqrt(deg)
    h = jnp.dot(z_ref[...], w1_ref[...], preferred_element_type=jnp.float32)
    h = h * dinv[:, None]
    dinv_ref[...] = dinv
    h0_ref[...] = h[:, :HALF]
    h1_ref[...] = h[:, HALF:]


def _tc_mid_body(a0_ref, a1_ref, h0_ref, h1_ref, dinv_ref, b_ref, w_ref,
                 g0_ref, g1_ref):
    dinv = dinv_ref[...]
    b = b_ref[...]
    x0 = jnp.tanh(dinv[:, None] * (a0_ref[...] + h0_ref[...]) + b[:HALF])
    x1 = jnp.tanh(dinv[:, None] * (a1_ref[...] + h1_ref[...]) + b[HALF:])
    x = jnp.concatenate([x0, x1], axis=1)
    g = jnp.dot(x, w_ref[...], preferred_element_type=jnp.float32)
    g = g * dinv[:, None]
    g0_ref[...] = g[:, :HALF]
    g1_ref[...] = g[:, HALF:]


def _tc_fin_body(a0_ref, a1_ref, h0_ref, h1_ref, dinv_ref, b_ref, wl_ref,
                 bl_ref, y_ref):
    dinv = dinv_ref[...]
    b = b_ref[...]
    x0 = jnp.tanh(dinv[:, None] * (a0_ref[...] + h0_ref[...]) + b[:HALF])
    x1 = jnp.tanh(dinv[:, None] * (a1_ref[...] + h1_ref[...]) + b[HALF:])
    x = jnp.concatenate([x0, x1], axis=1)
    y_ref[...] = (jnp.dot(x, wl_ref[...], preferred_element_type=jnp.float32)
                  + bl_ref[...])


_tc1 = pl.pallas_call(
    _tc1_body,
    out_shape=[
        jax.ShapeDtypeStruct((N,), jnp.float32),
        jax.ShapeDtypeStruct((N, HALF), jnp.float32),
        jax.ShapeDtypeStruct((N, HALF), jnp.float32),
    ],
)

_tc_mid = pl.pallas_call(
    _tc_mid_body,
    out_shape=[
        jax.ShapeDtypeStruct((N, HALF), jnp.float32),
        jax.ShapeDtypeStruct((N, HALF), jnp.float32),
    ],
)

_tc_fin = pl.pallas_call(
    _tc_fin_body,
    out_shape=jax.ShapeDtypeStruct((N, 1), jnp.float32),
)


def kernel(z, edge_index, W1, b1, W2, b2, Wl, bl):
    src = edge_index[0].astype(jnp.int32)
    dst = edge_index[1].astype(jnp.int32)
    zeros2d = jnp.zeros((N, HALF), jnp.float32)
    zeros_deg = jnp.zeros((N, 16), jnp.float32)
    ones_deg = jnp.ones((DEG_W, 16), jnp.float32)

    degp = _sc_degree(dst, zeros_deg, ones_deg)
    dinv, h0, h1 = _tc1(degp, z, W1)
    a0, a1 = _sc_scatter(h0, h1, src, dst, zeros2d)
    g0, g1 = _tc_mid(a0, a1, h0, h1, dinv, b1, W2)
    c0, c1 = _sc_scatter(g0, g1, src, dst, zeros2d)
    return _tc_fin(c0, c1, g0, g1, dinv, b2, Wl, bl)


# speedup vs baseline: 7.6251x; 7.6251x over previous
import functools

import jax
import jax.numpy as jnp
from jax import lax
from jax.experimental import pallas as pl
from jax.experimental.pallas import tpu as pltpu
from jax.experimental.pallas import tpu_sc as plsc

N = 10000
F = 256
HALF = 128
E = 160000
NC = 2
NS = 16
DEG_W = 40
DEG_EPT = E // (NC * NS)
SC_W = 80
SC_EPT = E // NS
ROWS_PT = 624
ROWS_TAIL = N - NS * ROWS_PT

@functools.cache
def _mesh():
    return plsc.VectorSubcoreMesh(
        core_axis_name="c", subcore_axis_name="s",
        num_cores=NC, num_subcores=NS)


@functools.cache
def _sc_degree_call():
    return pl.kernel(
        _sc_degree_body,
        out_type=jax.ShapeDtypeStruct((NC, N, HALF), jnp.float32),
        mesh=_mesh(),
        scratch_types=[
            pltpu.VMEM((DEG_W,), jnp.int32),
            pltpu.VMEM((DEG_W, HALF), jnp.float32),
            pltpu.VMEM_SHARED((N, HALF), jnp.float32),
        ],
    )


def _sc_degree_body(dst_hbm, zeros_hbm, ones_hbm, out_hbm, dst_v, ones_v, deg_sh):
    cid = lax.axis_index("c")
    sid = lax.axis_index("s")
    wid = cid * NS + sid

    pltpu.sync_copy(ones_hbm, ones_v)

    @pl.when(sid == 0)
    def _zero():
        pltpu.sync_copy(zeros_hbm, deg_sh)

    plsc.subcore_barrier()

    def win(w, carry):
        base = wid * DEG_EPT + w * DEG_W
        pltpu.sync_copy(dst_hbm.at[pl.ds(base, DEG_W)], dst_v)
        pltpu.sync_copy(ones_v, deg_sh.at[dst_v], add=True)
        return carry

    lax.fori_loop(0, DEG_EPT // DEG_W, win, 0)
    plsc.subcore_barrier()

    @pl.when(sid == 0)
    def _out():
        pltpu.sync_copy(deg_sh, out_hbm.at[cid])


@functools.cache
def _sc_scatter_call():
    return pl.kernel(
        _sc_scatter_body,
        out_type=[
            jax.ShapeDtypeStruct((N, HALF), jnp.float32),
            jax.ShapeDtypeStruct((N, HALF), jnp.float32),
        ],
        mesh=_mesh(),
        scratch_types=[
            pltpu.VMEM((SC_W,), jnp.int32),
            pltpu.VMEM((SC_W,), jnp.int32),
            pltpu.VMEM((SC_W, HALF), jnp.float32),
            pltpu.VMEM_SHARED((N, HALF), jnp.float32),
            pltpu.SemaphoreType.DMA,
        ],
    )


def _sc_scatter_body(h0_hbm, h1_hbm, src_hbm, dst_hbm, zeros_hbm, out0_hbm,
                     out1_hbm, src_v, dst_v, gath_v, acc_sh, sem):
    cid = lax.axis_index("c")
    sid = lax.axis_index("s")

    def run(h_hbm, out_hbm):
        @pl.when(sid < 10)
        def _zero():
            r0 = pl.multiple_of(sid * 1000, 8)
            pltpu.sync_copy(zeros_hbm.at[pl.ds(r0, 1000)],
                            acc_sh.at[pl.ds(r0, 1000)])
        plsc.subcore_barrier()

        def win(w, carry):
            base = sid * SC_EPT + w * SC_W
            pltpu.sync_copy(src_hbm.at[pl.ds(base, SC_W)], src_v)
            pltpu.sync_copy(dst_hbm.at[pl.ds(base, SC_W)], dst_v)
            pltpu.async_copy(h_hbm.at[src_v], gath_v, sem).wait()
            pltpu.sync_copy(gath_v, acc_sh.at[dst_v], add=True)
            return carry

        lax.fori_loop(0, SC_EPT // SC_W, win, 0)
        plsc.subcore_barrier()
        r0 = pl.multiple_of(sid * ROWS_PT, 8)
        pltpu.sync_copy(acc_sh.at[pl.ds(r0, ROWS_PT)],
                        out_hbm.at[pl.ds(r0, ROWS_PT)])

        @pl.when(sid == 0)
        def _tail():
            t0 = pl.multiple_of(NS * ROWS_PT, 8)
            pltpu.sync_copy(acc_sh.at[pl.ds(t0, ROWS_TAIL)],
                            out_hbm.at[pl.ds(t0, ROWS_TAIL)])

    @pl.when(cid == 0)
    def _c0():
        run(h0_hbm, out0_hbm)

    @pl.when(cid == 1)
    def _c1():
        run(h1_hbm, out1_hbm)


def _tc1_body(degp_ref, z_ref, w1_ref, dinv_ref, h0_ref, h1_ref):
    deg = jnp.sum(degp_ref[...], axis=(0, 2)) + 1.0
    dinv = lax.rsqrt(deg)
    h = jnp.dot(z_ref[...], w1_ref[...], preferred_element_type=jnp.float32)
    h = h * dinv[:, None]
    dinv_ref[...] = dinv
    h0_ref[...] = h[:, :HALF]
    h1_ref[...] = h[:, HALF:]


def _tc_mid_body(a0_ref, a1_ref, h0_ref, h1_ref, dinv_ref, b_ref, w_ref,
                 g0_ref, g1_ref):
    dinv = dinv_ref[...]
    b = b_ref[...]
    x0 = jnp.tanh(dinv[:, None] * (a0_ref[...] + h0_ref[...]) + b[:HALF])
    x1 = jnp.tanh(dinv[:, None] * (a1_ref[...] + h1_ref[...]) + b[HALF:])
    x = jnp.concatenate([x0, x1], axis=1)
    g = jnp.dot(x, w_ref[...], preferred_element_type=jnp.float32)
    g = g * dinv[:, None]
    g0_ref[...] = g[:, :HALF]
    g1_ref[...] = g[:, HALF:]


def _tc_fin_body(a0_ref, a1_ref, h0_ref, h1_ref, dinv_ref, b_ref, wl_ref,
                 bl_ref, y_ref):
    dinv = dinv_ref[...]
    b = b_ref[...]
    x0 = jnp.tanh(dinv[:, None] * (a0_ref[...] + h0_ref[...]) + b[:HALF])
    x1 = jnp.tanh(dinv[:, None] * (a1_ref[...] + h1_ref[...]) + b[HALF:])
    x = jnp.concatenate([x0, x1], axis=1)
    y_ref[...] = (jnp.dot(x, wl_ref[...], preferred_element_type=jnp.float32)
                  + bl_ref[...])


_tc1 = pl.pallas_call(
    _tc1_body,
    out_shape=[
        jax.ShapeDtypeStruct((N,), jnp.float32),
        jax.ShapeDtypeStruct((N, HALF), jnp.float32),
        jax.ShapeDtypeStruct((N, HALF), jnp.float32),
    ],
)

_tc_mid = pl.pallas_call(
    _tc_mid_body,
    out_shape=[
        jax.ShapeDtypeStruct((N, HALF), jnp.float32),
        jax.ShapeDtypeStruct((N, HALF), jnp.float32),
    ],
)

_tc_fin = pl.pallas_call(
    _tc_fin_body,
    out_shape=jax.ShapeDtypeStruct((N, 1), jnp.float32),
)


def kernel(z, edge_index, W1, b1, W2, b2, Wl, bl):
    src = edge_index[0].astype(jnp.int32)
    dst = edge_index[1].astype(jnp.int32)
    zeros2d = jnp.zeros((N, HALF), jnp.float32)
    zeros_deg = jnp.zeros((N, HALF), jnp.float32)
    ones_deg = jnp.full((DEG_W, HALF), 1.0 / HALF, jnp.float32)

    degp = _sc_degree_call()(dst, zeros_deg, ones_deg)
    dinv, h0, h1 = _tc1(degp, z, W1)
    a0, a1 = _sc_scatter_call()(h0, h1, src, dst, zeros2d)
    g0, g1 = _tc_mid(a0, a1, h0, h1, dinv, b1, W2)
    c0, c1 = _sc_scatter_call()(g0, g1, src, dst, zeros2d)
    return _tc_fin(c0, c1, g0, g1, dinv, b2, Wl, bl)


# speedup vs baseline: 16.3331x; 2.1420x over previous
import functools

import jax
import jax.numpy as jnp
from jax import lax
from jax.experimental import pallas as pl
from jax.experimental.pallas import tpu as pltpu
from jax.experimental.pallas import tpu_sc as plsc

N = 10000
F = 256
HALF = 128
E = 160000
NC = 2
NS = 16
DEG_W = 40
DEG_EPT = E // (NC * NS)
SC_W = 80
SC_EPT = E // NS
ROWS_PT = 624
ROWS_TAIL = N - NS * ROWS_PT

@functools.cache
def _mesh():
    return plsc.VectorSubcoreMesh(
        core_axis_name="c", subcore_axis_name="s",
        num_cores=NC, num_subcores=NS)


@functools.cache
def _sc_degree_call():
    return pl.kernel(
        _sc_degree_body,
        out_type=jax.ShapeDtypeStruct((NC, N, HALF), jnp.float32),
        mesh=_mesh(),
        scratch_types=[
            pltpu.VMEM((DEG_W,), jnp.int32),
            pltpu.VMEM((DEG_W, HALF), jnp.float32),
            pltpu.VMEM_SHARED((N, HALF), jnp.float32),
        ],
    )


def _sc_degree_body(dst_hbm, zeros_hbm, ones_hbm, out_hbm, dst_v, ones_v, deg_sh):
    cid = lax.axis_index("c")
    sid = lax.axis_index("s")
    wid = cid * NS + sid

    pltpu.sync_copy(ones_hbm, ones_v)

    @pl.when(sid == 0)
    def _zero():
        pltpu.sync_copy(zeros_hbm, deg_sh)

    plsc.subcore_barrier()

    def win(w, carry):
        base = wid * DEG_EPT + w * DEG_W
        pltpu.sync_copy(dst_hbm.at[pl.ds(base, DEG_W)], dst_v)
        pltpu.sync_copy(ones_v, deg_sh.at[dst_v], add=True)
        return carry

    lax.fori_loop(0, DEG_EPT // DEG_W, win, 0)
    plsc.subcore_barrier()

    @pl.when(sid == 0)
    def _out():
        pltpu.sync_copy(deg_sh, out_hbm.at[cid])


NBUF = 4
ITERS = SC_EPT // SC_W
RING_ITERS = -(-(ITERS + 2 - NBUF) // NBUF)


@functools.cache
def _sc_scatter_call():
    return pl.kernel(
        _sc_scatter_body,
        out_type=[
            jax.ShapeDtypeStruct((N, HALF), jnp.float32),
            jax.ShapeDtypeStruct((N, HALF), jnp.float32),
        ],
        mesh=_mesh(),
        scratch_types=(
            [pltpu.VMEM((2, SC_W), jnp.int32) for _ in range(NBUF)]
            + [pltpu.VMEM((SC_W, HALF), jnp.float32) for _ in range(NBUF)]
            + [pltpu.VMEM_SHARED((N, HALF), jnp.float32)]
            + [pltpu.SemaphoreType.DMA for _ in range(2 * NBUF)]
        ),
    )


def _sc_scatter_body(h0_hbm, h1_hbm, idx_hbm, zeros_hbm, out0_hbm,
                     out1_hbm, *scratch):
    idx_v = scratch[:NBUF]
    gath_v = scratch[NBUF:2 * NBUF]
    acc_sh = scratch[2 * NBUF]
    gsem = scratch[2 * NBUF + 1:3 * NBUF + 1]
    ssem = scratch[3 * NBUF + 1:]
    cid = lax.axis_index("c")
    sid = lax.axis_index("s")

    def run(h_hbm, out_hbm):
        @pl.when(sid < 10)
        def _zero():
            r0 = pl.multiple_of(sid * 1000, 8)
            pltpu.sync_copy(zeros_hbm.at[pl.ds(r0, 1000)],
                            acc_sh.at[pl.ds(r0, 1000)])
        plsc.subcore_barrier()

        def load_and_gather(g, b):
            pltpu.sync_copy(idx_hbm.at[sid, g], idx_v[b])
            pltpu.make_async_copy(h_hbm.at[idx_v[b].at[0]], gath_v[b],
                                  gsem[b]).start()

        def fire_scatter(b2):
            pltpu.make_async_copy(h_hbm.at[pl.ds(0, SC_W)], gath_v[b2],
                                  gsem[b2]).wait()
            pltpu.async_copy(gath_v[b2], acc_sh.at[idx_v[b2].at[1]],
                             ssem[b2], add=True)

        def drain_scatter(b):
            pltpu.make_async_copy(h_hbm.at[pl.ds(0, SC_W)], gath_v[b],
                                  ssem[b]).wait()

        for b in range(NBUF):
            load_and_gather(b, b)
            if b >= 2:
                fire_scatter(b - 2)

        def ring(G, carry):
            g0 = NBUF + G * NBUF
            for b in range(NBUF):
                g = g0 + b

                @pl.when(g - NBUF < ITERS - 1)
                def _dr():
                    drain_scatter(b)

                @pl.when(g < ITERS)
                def _lg():
                    load_and_gather(g, b)

                @pl.when(g - 2 < ITERS)
                def _fs():
                    fire_scatter((b + 2) % NBUF)
            return carry

        lax.fori_loop(0, RING_ITERS, ring, 0)
        drain_scatter((ITERS - 1) % NBUF)
        plsc.subcore_barrier()
        r0 = pl.multiple_of(sid * ROWS_PT, 8)
        pltpu.sync_copy(acc_sh.at[pl.ds(r0, ROWS_PT)],
                        out_hbm.at[pl.ds(r0, ROWS_PT)])

        @pl.when(sid == 0)
        def _tail():
            t0 = pl.multiple_of(NS * ROWS_PT, 8)
            pltpu.sync_copy(acc_sh.at[pl.ds(t0, ROWS_TAIL)],
                            out_hbm.at[pl.ds(t0, ROWS_TAIL)])

    @pl.when(cid == 0)
    def _c0():
        run(h0_hbm, out0_hbm)

    @pl.when(cid == 1)
    def _c1():
        run(h1_hbm, out1_hbm)


def _tc1_body(degp_ref, z_ref, w1_ref, dinv_ref, h0_ref, h1_ref):
    deg = jnp.sum(degp_ref[...], axis=(0, 2)) + 1.0
    dinv = lax.rsqrt(deg)
    h = jnp.dot(z_ref[...], w1_ref[...], preferred_element_type=jnp.float32)
    h = h * dinv[:, None]
    dinv_ref[...] = dinv
    h0_ref[...] = h[:, :HALF]
    h1_ref[...] = h[:, HALF:]


def _tc_mid_body(a0_ref, a1_ref, h0_ref, h1_ref, dinv_ref, b_ref, w_ref,
                 g0_ref, g1_ref):
    dinv = dinv_ref[...]
    b = b_ref[...]
    x0 = jnp.tanh(dinv[:, None] * (a0_ref[...] + h0_ref[...]) + b[:HALF])
    x1 = jnp.tanh(dinv[:, None] * (a1_ref[...] + h1_ref[...]) + b[HALF:])
    x = jnp.concatenate([x0, x1], axis=1)
    g = jnp.dot(x, w_ref[...], preferred_element_type=jnp.float32)
    g = g * dinv[:, None]
    g0_ref[...] = g[:, :HALF]
    g1_ref[...] = g[:, HALF:]


def _tc_fin_body(a0_ref, a1_ref, h0_ref, h1_ref, dinv_ref, b_ref, wl_ref,
                 bl_ref, y_ref):
    dinv = dinv_ref[...]
    b = b_ref[...]
    x0 = jnp.tanh(dinv[:, None] * (a0_ref[...] + h0_ref[...]) + b[:HALF])
    x1 = jnp.tanh(dinv[:, None] * (a1_ref[...] + h1_ref[...]) + b[HALF:])
    x = jnp.concatenate([x0, x1], axis=1)
    y_ref[...] = (jnp.dot(x, wl_ref[...], preferred_element_type=jnp.float32)
                  + bl_ref[...])


_tc1 = pl.pallas_call(
    _tc1_body,
    out_shape=[
        jax.ShapeDtypeStruct((N,), jnp.float32),
        jax.ShapeDtypeStruct((N, HALF), jnp.float32),
        jax.ShapeDtypeStruct((N, HALF), jnp.float32),
    ],
)

_tc_mid = pl.pallas_call(
    _tc_mid_body,
    out_shape=[
        jax.ShapeDtypeStruct((N, HALF), jnp.float32),
        jax.ShapeDtypeStruct((N, HALF), jnp.float32),
    ],
)

_tc_fin = pl.pallas_call(
    _tc_fin_body,
    out_shape=jax.ShapeDtypeStruct((N, 1), jnp.float32),
)


def kernel(z, edge_index, W1, b1, W2, b2, Wl, bl):
    src = edge_index[0].astype(jnp.int32)
    dst = edge_index[1].astype(jnp.int32)
    zeros2d = jnp.zeros((N, HALF), jnp.float32)
    zeros_deg = jnp.zeros((N, HALF), jnp.float32)
    ones_deg = jnp.full((DEG_W, HALF), 1.0 / HALF, jnp.float32)

    idx = jnp.stack([src.reshape(NS, ITERS, SC_W),
                     dst.reshape(NS, ITERS, SC_W)], axis=2)

    degp = _sc_degree_call()(dst, zeros_deg, ones_deg)
    dinv, h0, h1 = _tc1(degp, z, W1)
    a0, a1 = _sc_scatter_call()(h0, h1, idx, zeros2d)
    g0, g1 = _tc_mid(a0, a1, h0, h1, dinv, b1, W2)
    c0, c1 = _sc_scatter_call()(g0, g1, idx, zeros2d)
    return _tc_fin(c0, c1, g0, g1, dinv, b2, Wl, bl)


# speedup vs baseline: 19.0679x; 1.1674x over previous
import functools

import jax
import jax.numpy as jnp
from jax import lax
from jax.experimental import pallas as pl
from jax.experimental.pallas import tpu as pltpu
from jax.experimental.pallas import tpu_sc as plsc

N = 10000
F = 256
HALF = 128
E = 160000
NC = 2
NS = 16
DEG_W = 100
DEG_EPT = E // (NC * NS)
DEG_ITERS = DEG_EPT // DEG_W
SC_W = 80
SC_EPT = E // NS
ROWS_PT = 624
ROWS_TAIL = N - NS * ROWS_PT

@functools.cache
def _mesh():
    return plsc.VectorSubcoreMesh(
        core_axis_name="c", subcore_axis_name="s",
        num_cores=NC, num_subcores=NS)


@functools.cache
def _sc_degree_call():
    return pl.kernel(
        _sc_degree_body,
        out_type=jax.ShapeDtypeStruct((NC, N, HALF), jnp.float32),
        mesh=_mesh(),
        scratch_types=(
            [pltpu.VMEM((DEG_W,), jnp.int32) for _ in range(NBUF)]
            + [pltpu.VMEM((DEG_W, HALF), jnp.float32),
               pltpu.VMEM_SHARED((N, HALF), jnp.float32)]
            + [pltpu.SemaphoreType.DMA for _ in range(NBUF)]
        ),
    )


def _sc_degree_body(dst_hbm, zeros_hbm, ones_hbm, out_hbm, *scratch):
    dst_v = scratch[:NBUF]
    ones_v = scratch[NBUF]
    deg_sh = scratch[NBUF + 1]
    ssem = scratch[NBUF + 2:]
    cid = lax.axis_index("c")
    sid = lax.axis_index("s")
    wid = cid * NS + sid

    pltpu.sync_copy(ones_hbm, ones_v)

    @pl.when(sid == 0)
    def _zero():
        pltpu.sync_copy(zeros_hbm, deg_sh)

    plsc.subcore_barrier()

    def load_and_fire(g, b):
        pltpu.sync_copy(dst_hbm.at[wid, g], dst_v[b])
        pltpu.async_copy(ones_v, deg_sh.at[dst_v[b]], ssem[b], add=True)

    def drain(b):
        pltpu.make_async_copy(ones_hbm, ones_v, ssem[b]).wait()

    for b in range(min(NBUF, DEG_ITERS)):
        load_and_fire(b, b)

    def ring(G, carry):
        g0 = NBUF + G * NBUF
        for b in range(NBUF):
            g = g0 + b

            @pl.when(g - NBUF < DEG_ITERS)
            def _dr():
                drain(b)

            @pl.when(g < DEG_ITERS)
            def _lf():
                load_and_fire(g, b)
        return carry

    ring_iters = -(-(DEG_ITERS - NBUF) // NBUF)
    lax.fori_loop(0, ring_iters, ring, 0)
    g_max = NBUF + ring_iters * NBUF - 1
    for w in range(max(0, g_max - NBUF + 1), DEG_ITERS):
        drain(w % NBUF)
    plsc.subcore_barrier()

    @pl.when(sid == 0)
    def _out():
        pltpu.sync_copy(deg_sh, out_hbm.at[cid])


NBUF = 4
ITERS = SC_EPT // SC_W
RING_ITERS = -(-(ITERS + 2 - NBUF) // NBUF)


@functools.cache
def _sc_scatter_call():
    return pl.kernel(
        _sc_scatter_body,
        out_type=[
            jax.ShapeDtypeStruct((N, HALF), jnp.float32),
            jax.ShapeDtypeStruct((N, HALF), jnp.float32),
        ],
        mesh=_mesh(),
        scratch_types=(
            [pltpu.VMEM((2, SC_W), jnp.int32) for _ in range(NBUF)]
            + [pltpu.VMEM((SC_W, HALF), jnp.float32) for _ in range(NBUF)]
            + [pltpu.VMEM_SHARED((N, HALF), jnp.float32)]
            + [pltpu.SemaphoreType.DMA for _ in range(2 * NBUF)]
        ),
    )


def _sc_scatter_body(h0_hbm, h1_hbm, idx_hbm, zeros_hbm, out0_hbm,
                     out1_hbm, *scratch):
    idx_v = scratch[:NBUF]
    gath_v = scratch[NBUF:2 * NBUF]
    acc_sh = scratch[2 * NBUF]
    gsem = scratch[2 * NBUF + 1:3 * NBUF + 1]
    ssem = scratch[3 * NBUF + 1:]
    cid = lax.axis_index("c")
    sid = lax.axis_index("s")

    def run(h_hbm, out_hbm):
        @pl.when(sid < 10)
        def _zero():
            r0 = pl.multiple_of(sid * 1000, 8)
            pltpu.sync_copy(zeros_hbm.at[pl.ds(r0, 1000)],
                            acc_sh.at[pl.ds(r0, 1000)])
        plsc.subcore_barrier()

        def load_and_gather(g, b):
            pltpu.sync_copy(idx_hbm.at[sid, g], idx_v[b])
            pltpu.make_async_copy(h_hbm.at[idx_v[b].at[0]], gath_v[b],
                                  gsem[b]).start()

        def fire_scatter(b2):
            pltpu.make_async_copy(h_hbm.at[pl.ds(0, SC_W)], gath_v[b2],
                                  gsem[b2]).wait()
            pltpu.async_copy(gath_v[b2], acc_sh.at[idx_v[b2].at[1]],
                             ssem[b2], add=True)

        def drain_scatter(b):
            pltpu.make_async_copy(h_hbm.at[pl.ds(0, SC_W)], gath_v[b],
                                  ssem[b]).wait()

        for b in range(NBUF):
            load_and_gather(b, b)
            if b >= 2:
                fire_scatter(b - 2)

        def ring(G, carry):
            g0 = NBUF + G * NBUF
            for b in range(NBUF):
                g = g0 + b

                @pl.when(g - NBUF < ITERS - 1)
                def _dr():
                    drain_scatter(b)

                @pl.when(g < ITERS)
                def _lg():
                    load_and_gather(g, b)

                @pl.when(g - 2 < ITERS)
                def _fs():
                    fire_scatter((b + 2) % NBUF)
            return carry

        lax.fori_loop(0, RING_ITERS, ring, 0)
        drain_scatter((ITERS - 1) % NBUF)
        plsc.subcore_barrier()
        r0 = pl.multiple_of(sid * ROWS_PT, 8)
        pltpu.sync_copy(acc_sh.at[pl.ds(r0, ROWS_PT)],
                        out_hbm.at[pl.ds(r0, ROWS_PT)])

        @pl.when(sid == 0)
        def _tail():
            t0 = pl.multiple_of(NS * ROWS_PT, 8)
            pltpu.sync_copy(acc_sh.at[pl.ds(t0, ROWS_TAIL)],
                            out_hbm.at[pl.ds(t0, ROWS_TAIL)])

    @pl.when(cid == 0)
    def _c0():
        run(h0_hbm, out0_hbm)

    @pl.when(cid == 1)
    def _c1():
        run(h1_hbm, out1_hbm)


def _tc1_body(degp_ref, z_ref, w1_ref, dinv_ref, h0_ref, h1_ref):
    deg = jnp.sum(degp_ref[...], axis=(0, 2)) + 1.0
    dinv = lax.rsqrt(deg)
    h = jnp.dot(z_ref[...], w1_ref[...], preferred_element_type=jnp.float32)
    h = h * dinv[:, None]
    dinv_ref[...] = dinv
    h0_ref[...] = h[:, :HALF]
    h1_ref[...] = h[:, HALF:]


def _tc_mid_body(a0_ref, a1_ref, h0_ref, h1_ref, dinv_ref, b_ref, w_ref,
                 g0_ref, g1_ref):
    dinv = dinv_ref[...]
    b = b_ref[...]
    x0 = jnp.tanh(dinv[:, None] * (a0_ref[...] + h0_ref[...]) + b[:HALF])
    x1 = jnp.tanh(dinv[:, None] * (a1_ref[...] + h1_ref[...]) + b[HALF:])
    x = jnp.concatenate([x0, x1], axis=1)
    g = jnp.dot(x, w_ref[...], preferred_element_type=jnp.float32)
    g = g * dinv[:, None]
    g0_ref[...] = g[:, :HALF]
    g1_ref[...] = g[:, HALF:]


def _tc_fin_body(a0_ref, a1_ref, h0_ref, h1_ref, dinv_ref, b_ref, wl_ref,
                 bl_ref, y_ref):
    dinv = dinv_ref[...]
    b = b_ref[...]
    x0 = jnp.tanh(dinv[:, None] * (a0_ref[...] + h0_ref[...]) + b[:HALF])
    x1 = jnp.tanh(dinv[:, None] * (a1_ref[...] + h1_ref[...]) + b[HALF:])
    x = jnp.concatenate([x0, x1], axis=1)
    y_ref[...] = (jnp.dot(x, wl_ref[...], preferred_element_type=jnp.float32)
                  + bl_ref[...])


_tc1 = pl.pallas_call(
    _tc1_body,
    out_shape=[
        jax.ShapeDtypeStruct((N,), jnp.float32),
        jax.ShapeDtypeStruct((N, HALF), jnp.float32),
        jax.ShapeDtypeStruct((N, HALF), jnp.float32),
    ],
)

_tc_mid = pl.pallas_call(
    _tc_mid_body,
    out_shape=[
        jax.ShapeDtypeStruct((N, HALF), jnp.float32),
        jax.ShapeDtypeStruct((N, HALF), jnp.float32),
    ],
)

_tc_fin = pl.pallas_call(
    _tc_fin_body,
    out_shape=jax.ShapeDtypeStruct((N, 1), jnp.float32),
)


def kernel(z, edge_index, W1, b1, W2, b2, Wl, bl):
    src = edge_index[0].astype(jnp.int32)
    dst = edge_index[1].astype(jnp.int32)
    zeros2d = jnp.zeros((N, HALF), jnp.float32)
    zeros_deg = jnp.zeros((N, HALF), jnp.float32)
    ones_deg = jnp.full((DEG_W, HALF), 1.0 / HALF, jnp.float32)

    idx = jnp.stack([src.reshape(NS, ITERS, SC_W),
                     dst.reshape(NS, ITERS, SC_W)], axis=2)

    dst_deg = dst.reshape(NC * NS, DEG_ITERS, DEG_W)
    degp = _sc_degree_call()(dst_deg, zeros_deg, ones_deg)
    dinv, h0, h1 = _tc1(degp, z, W1)
    a0, a1 = _sc_scatter_call()(h0, h1, idx, zeros2d)
    g0, g1 = _tc_mid(a0, a1, h0, h1, dinv, b1, W2)
    c0, c1 = _sc_scatter_call()(g0, g1, idx, zeros2d)
    return _tc_fin(c0, c1, g0, g1, dinv, b2, Wl, bl)
